# baseline (speedup 1.0000x reference)
import functools

import numpy as np
import jax
import jax.numpy as jnp
from jax import lax
from jax.experimental import pallas as pl
from jax.experimental.pallas import tpu as pltpu
from jax.experimental.pallas import tpu_sc as plsc

NUM_LEVELS = 16
TABLE_SIZE = 2 ** 19
MASK = TABLE_SIZE - 1
FEAT = 2
N = 262144
NC = 2
NS = 16
NW = NC * NS
PER_TILE = N // NW
C = 512
N_CHUNKS = PER_TILE // C
NGROUP = C // 16
GXFER = 128
NXFER = 8 * C // GXFER


def _i32(v: int) -> int:
    v &= 0xFFFFFFFF
    return v - (1 << 32) if v >= (1 << 31) else v


_P1 = _i32(2654435761)
_P2 = _i32(805459861)
_P3 = 3674653429
_SCALES = [float(np.float32(np.float64(1.5) ** l)) for l in range(NUM_LEVELS)]
_KLEV = [_i32(_P3 * l) for l in range(NUM_LEVELS)]


def _hash_encode_sc(x, tab):
    mesh = plsc.VectorSubcoreMesh(core_axis_name="c", subcore_axis_name="s")

    @functools.partial(
        pl.kernel,
        out_type=jax.ShapeDtypeStruct((N * NUM_LEVELS * FEAT,), jnp.float32),
        mesh=mesh,
        compiler_params=pltpu.CompilerParams(needs_layout_passes=False,
                                             use_tc_tiling_on_sc=False),
        scratch_types=[
            pltpu.VMEM((C, 3), jnp.float32),
            pltpu.VMEM((C * NUM_LEVELS * FEAT,), jnp.float32),
            pltpu.VMEM((NXFER, GXFER), jnp.int32),
            pltpu.VMEM((NXFER, GXFER), jnp.int32),
            pltpu.VMEM((NXFER, GXFER), jnp.int32),
            pltpu.VMEM((NXFER, GXFER), jnp.int32),
            pltpu.VMEM((8 * C,), jnp.float32),
            pltpu.VMEM((8 * C,), jnp.float32),
            pltpu.VMEM((8 * C,), jnp.float32),
            pltpu.VMEM((8 * C,), jnp.float32),
            pltpu.SemaphoreType.DMA,
            pltpu.SemaphoreType.DMA,
        ],
    )
    def body(x_hbm, tab_hbm, out_hbm,
             x_v, out_v, ia0, ib0, ia1, ib1, ra0, rb0, ra1, rb1, sem0, sem1):
        i32c = jnp.int32
        wid = lax.axis_index("s") * i32c(NC) + lax.axis_index("c")
        tile_base = wid * i32c(PER_TILE)
        iota = lax.iota(jnp.int32, 16)
        zero_f = jnp.zeros((16,), jnp.float32)
        one_f = jnp.full((16,), 1.0, jnp.float32)
        idx_bufs = ((ia0, ib0), (ia1, ib1))
        rows_bufs = ((ra0, rb0), (ra1, rb1))
        sems = (sem0, sem1)

        def load_x(g, d):
            ridx = jnp.full((16,), g * i32c(16), jnp.int32) + iota
            return plsc.load_gather(x_v, [ridx, jnp.full((16,), d, jnp.int32)])

        def scaled(g, l):
            sc = jnp.full((16,), _SCALES[l], jnp.float32)
            out = []
            for d in range(3):
                xd = load_x(g, d)
                xd = jnp.minimum(jnp.maximum(xd, zero_f), one_f)
                out.append(xd * sc)
            return out

        def pass_a(l, idx_refs):
            klev = jnp.full((16,), _KLEV[l], jnp.int32)
            p1 = jnp.full((16,), _P1, jnp.int32)
            p2 = jnp.full((16,), _P2, jnp.int32)
            mask = jnp.full((16,), MASK, jnp.int32)
            base_l = jnp.full((16,), l * TABLE_SIZE * FEAT, jnp.int32)
            one_i = jnp.full((16,), 1, jnp.int32)

            def g_body(g, _):
                sx = scaled(g, l)
                xi = [s.astype(jnp.int32) for s in sx]
                a0 = xi[0]
                a1 = xi[0] + one_i
                mm1 = xi[1] * p1
                m1 = (mm1, mm1 + p1)
                mm2 = xi[2] * p2
                m2 = (mm2 ^ klev, (mm2 + p2) ^ klev)
                t = ((a0 ^ m1[0], a0 ^ m1[1]), (a1 ^ m1[0], a1 ^ m1[1]))
                row = g >> i32c(3)
                col = (g & i32c(7)) * i32c(16)
                for c in range(8):
                    b0, b1, b2 = c & 1, (c >> 1) & 1, (c >> 2) & 1
                    h = (t[b0][b1] ^ m2[b2]) & mask
                    e0 = h + h + base_l
                    r = i32c(c * (C // GXFER)) + row
                    idx_refs[0][r, pl.ds(col, 16)] = e0
                    idx_refs[1][r, pl.ds(col, 16)] = e0 + one_i
                return jnp.int32(0)

            lax.fori_loop(jnp.int32(0), jnp.int32(NGROUP), g_body, jnp.int32(0))

        def fire(idx_refs, rows_refs, sem):
            def f_body(j, _):
                off = j * i32c(GXFER)
                for f in range(FEAT):
                    pltpu.async_copy(
                        tab_hbm.at[idx_refs[f].at[j]],
                        rows_refs[f].at[pl.ds(off, GXFER)],
                        sem)
                return jnp.int32(0)

            lax.fori_loop(jnp.int32(0), jnp.int32(NXFER), f_body, jnp.int32(0))

        def drain(idx_refs, rows_refs, sem):
            def d_body(j, _):
                off = j * i32c(GXFER)
                for f in range(FEAT):
                    pltpu.make_async_copy(
                        tab_hbm.at[idx_refs[f].at[j]],
                        rows_refs[f].at[pl.ds(off, GXFER)],
                        sem).wait()
                return jnp.int32(0)

            lax.fori_loop(jnp.int32(0), jnp.int32(NXFER), d_body, jnp.int32(0))

        def pass_b(l, rows_refs):
            def g_body(g, _):
                sx = scaled(g, l)
                xf = [s - s.astype(jnp.int32).astype(jnp.float32) for s in sx]
                w0 = [one_f - f for f in xf]
                wxy = ((w0[0] * w0[1], w0[0] * xf[1]),
                       (xf[0] * w0[1], xf[0] * xf[1]))
                wz = (w0[2], xf[2])
                row0 = jnp.full((16,), g * i32c(16), jnp.int32) + iota
                opos = row0 * jnp.full((16,), NUM_LEVELS * FEAT, jnp.int32)
                acc0 = zero_f
                acc1 = zero_f
                for c in range(8):
                    b0, b1, b2 = c & 1, (c >> 1) & 1, (c >> 2) & 1
                    w = wxy[b0][b1] * wz[b2]
                    ridx = row0 + jnp.full((16,), c * C, jnp.int32)
                    f0 = plsc.load_gather(rows_refs[0], [ridx])
                    f1 = plsc.load_gather(rows_refs[1], [ridx])
                    acc0 = acc0 + w * f0
                    acc1 = acc1 + w * f1
                plsc.store_scatter(
                    out_v, [opos + jnp.full((16,), 2 * l, jnp.int32)], acc0)
                plsc.store_scatter(
                    out_v, [opos + jnp.full((16,), 2 * l + 1, jnp.int32)], acc1)
                return jnp.int32(0)

            lax.fori_loop(jnp.int32(0), jnp.int32(NGROUP), g_body, jnp.int32(0))

        def chunk_body(ch, _):
            base = tile_base + ch * i32c(C)
            pltpu.sync_copy(x_hbm.at[pl.ds(base, C), :], x_v)
            pass_a(0, idx_bufs[0])
            fire(idx_bufs[0], rows_bufs[0], sems[0])
            for l in range(NUM_LEVELS):
                b = l % 2
                nb = 1 - b
                if l + 1 < NUM_LEVELS:
                    pass_a(l + 1, idx_bufs[nb])
                    fire(idx_bufs[nb], rows_bufs[nb], sems[nb])
                drain(idx_bufs[b], rows_bufs[b], sems[b])
                pass_b(l, rows_bufs[b])
            pltpu.sync_copy(
                out_v,
                out_hbm.at[pl.ds(base * i32c(NUM_LEVELS * FEAT),
                                 C * NUM_LEVELS * FEAT)])
            return jnp.int32(0)

        lax.fori_loop(jnp.int32(0), jnp.int32(N_CHUNKS), chunk_body, jnp.int32(0))

    return body(x, tab)


def kernel(x, tables):
    tab = tables.reshape(NUM_LEVELS * TABLE_SIZE * FEAT)
    return _hash_encode_sc(x, tab).reshape(N, NUM_LEVELS * FEAT)


# speedup vs baseline: 1.0565x; 1.0565x over previous
import functools

import numpy as np
import jax
import jax.numpy as jnp
from jax import lax
from jax.experimental import pallas as pl
from jax.experimental.pallas import tpu as pltpu
from jax.experimental.pallas import tpu_sc as plsc

NUM_LEVELS = 16
TABLE_SIZE = 2 ** 19
MASK = TABLE_SIZE - 1
FEAT = 2
N = 262144
NC = 2
NS = 16
NW = NC * NS
PER_TILE = N // NW
C = 512
N_CHUNKS = PER_TILE // C
NGROUP = C // 16
GXFER = 128
NXFER = 8 * C // GXFER


def _i32(v: int) -> int:
    v &= 0xFFFFFFFF
    return v - (1 << 32) if v >= (1 << 31) else v


_P1 = _i32(2654435761)
_P2 = _i32(805459861)
_P3 = 3674653429
_SCALES = [float(np.float32(np.float64(1.5) ** l)) for l in range(NUM_LEVELS)]
_KLEV = [_i32(_P3 * l) for l in range(NUM_LEVELS)]


def _hash_encode_sc(x, tab):
    mesh = plsc.VectorSubcoreMesh(core_axis_name="c", subcore_axis_name="s")

    @functools.partial(
        pl.kernel,
        out_type=jax.ShapeDtypeStruct((N * NUM_LEVELS * FEAT,), jnp.float32),
        mesh=mesh,
        compiler_params=pltpu.CompilerParams(needs_layout_passes=False,
                                             use_tc_tiling_on_sc=False),
        scratch_types=[
            pltpu.VMEM((C, 3), jnp.float32),
            pltpu.VMEM((C * NUM_LEVELS * FEAT,), jnp.float32),
            pltpu.VMEM((NXFER, GXFER), jnp.int32),
            pltpu.VMEM((NXFER, GXFER), jnp.int32),
            pltpu.VMEM((NXFER, GXFER), jnp.int32),
            pltpu.VMEM((NXFER, GXFER), jnp.int32),
            pltpu.VMEM((8 * C,), jnp.float32),
            pltpu.VMEM((8 * C,), jnp.float32),
            pltpu.VMEM((8 * C,), jnp.float32),
            pltpu.VMEM((8 * C,), jnp.float32),
            pltpu.SemaphoreType.DMA,
            pltpu.SemaphoreType.DMA,
        ],
    )
    def body(x_hbm, tab_hbm, out_hbm,
             x_v, out_v, ia0, ib0, ia1, ib1, ra0, rb0, ra1, rb1, sem0, sem1):
        i32c = jnp.int32
        wid = lax.axis_index("s") * i32c(NC) + lax.axis_index("c")
        tile_base = wid * i32c(PER_TILE)
        iota = lax.iota(jnp.int32, 16)
        zero_f = jnp.zeros((16,), jnp.float32)
        one_f = jnp.full((16,), 1.0, jnp.float32)
        idx_bufs = ((ia0, ib0), (ia1, ib1))
        rows_bufs = ((ra0, rb0), (ra1, rb1))
        sems = (sem0, sem1)

        def load_x(g, d):
            ridx = jnp.full((16,), g * i32c(16), jnp.int32) + iota
            return plsc.load_gather(x_v, [ridx, jnp.full((16,), d, jnp.int32)])

        def scaled(g, l):
            sc = jnp.full((16,), _SCALES[l], jnp.float32)
            out = []
            for d in range(3):
                xd = load_x(g, d)
                xd = jnp.minimum(jnp.maximum(xd, zero_f), one_f)
                out.append(xd * sc)
            return out

        def pass_a(l, idx_refs):
            klev = jnp.full((16,), _KLEV[l], jnp.int32)
            p1 = jnp.full((16,), _P1, jnp.int32)
            p2 = jnp.full((16,), _P2, jnp.int32)
            mask = jnp.full((16,), MASK, jnp.int32)
            base_l = jnp.full((16,), l * TABLE_SIZE * FEAT, jnp.int32)
            one_i = jnp.full((16,), 1, jnp.int32)

            def g_body(g, _):
                sx = scaled(g, l)
                xi = [s.astype(jnp.int32) for s in sx]
                a0 = xi[0]
                a1 = xi[0] + one_i
                mm1 = xi[1] * p1
                m1 = (mm1, mm1 + p1)
                mm2 = xi[2] * p2
                m2 = (mm2 ^ klev, (mm2 + p2) ^ klev)
                t = ((a0 ^ m1[0], a0 ^ m1[1]), (a1 ^ m1[0], a1 ^ m1[1]))
                row = g >> i32c(3)
                col = (g & i32c(7)) * i32c(16)
                for c in range(8):
                    b0, b1, b2 = c & 1, (c >> 1) & 1, (c >> 2) & 1
                    h = (t[b0][b1] ^ m2[b2]) & mask
                    e0 = h + h + base_l
                    r = i32c(c * (C // GXFER)) + row
                    idx_refs[0][r, pl.ds(col, 16)] = e0
                    idx_refs[1][r, pl.ds(col, 16)] = e0 + one_i
                return jnp.int32(0)

            lax.fori_loop(jnp.int32(0), jnp.int32(NGROUP), g_body, jnp.int32(0))

        def fire(idx_refs, rows_refs, sem):
            def f_body(j, _):
                off = j * i32c(GXFER)
                for f in range(FEAT):
                    pltpu.async_copy(
                        tab_hbm.at[idx_refs[f].at[j]],
                        rows_refs[f].at[pl.ds(off, GXFER)],
                        sem)
                return jnp.int32(0)

            lax.fori_loop(jnp.int32(0), jnp.int32(NXFER), f_body, jnp.int32(0))

        def drain(idx_refs, rows_refs, sem):
            def d_body(j, _):
                off = j * i32c(GXFER)
                for f in range(FEAT):
                    pltpu.make_async_copy(
                        tab_hbm.at[idx_refs[f].at[j]],
                        rows_refs[f].at[pl.ds(off, GXFER)],
                        sem).wait()
                return jnp.int32(0)

            lax.fori_loop(jnp.int32(0), jnp.int32(NXFER), d_body, jnp.int32(0))

        def pass_b(l, rows_refs):
            def g_body(g, _):
                sx = scaled(g, l)
                xf = [s - s.astype(jnp.int32).astype(jnp.float32) for s in sx]
                w0 = [one_f - f for f in xf]
                wxy = ((w0[0] * w0[1], w0[0] * xf[1]),
                       (xf[0] * w0[1], xf[0] * xf[1]))
                wz = (w0[2], xf[2])
                row0 = jnp.full((16,), g * i32c(16), jnp.int32) + iota
                opos = row0 * jnp.full((16,), NUM_LEVELS * FEAT, jnp.int32)
                acc0 = zero_f
                acc1 = zero_f
                for c in range(8):
                    b0, b1, b2 = c & 1, (c >> 1) & 1, (c >> 2) & 1
                    w = wxy[b0][b1] * wz[b2]
                    ridx = row0 + jnp.full((16,), c * C, jnp.int32)
                    f0 = plsc.load_gather(rows_refs[0], [ridx])
                    f1 = plsc.load_gather(rows_refs[1], [ridx])
                    acc0 = acc0 + w * f0
                    acc1 = acc1 + w * f1
                plsc.store_scatter(
                    out_v, [opos + jnp.full((16,), 2 * l, jnp.int32)], acc0)
                plsc.store_scatter(
                    out_v, [opos + jnp.full((16,), 2 * l + 1, jnp.int32)], acc1)
                return jnp.int32(0)

            lax.fori_loop(jnp.int32(0), jnp.int32(NGROUP), g_body, jnp.int32(0))

        def chunk_body(ch, _):
            base = tile_base + ch * i32c(C)
            pltpu.sync_copy(x_hbm.at[pl.ds(base, C), :], x_v)
            pass_a(0, idx_bufs[0])
            fire(idx_bufs[0], rows_bufs[0], sems[0])
            for l in range(NUM_LEVELS):
                b = l % 2
                nb = 1 - b
                if l + 1 < NUM_LEVELS:
                    pass_a(l + 1, idx_bufs[nb])
                    fire(idx_bufs[nb], rows_bufs[nb], sems[nb])
                drain(idx_bufs[b], rows_bufs[b], sems[b])
                pass_b(l, rows_bufs[b])
            pltpu.sync_copy(
                out_v,
                out_hbm.at[pl.ds(base * i32c(NUM_LEVELS * FEAT),
                                 C * NUM_LEVELS * FEAT)])
            return jnp.int32(0)

        lax.fori_loop(jnp.int32(0), jnp.int32(N_CHUNKS), chunk_body, jnp.int32(0))

    return body(x, tab)


def kernel(x, tables):
    one = jnp.float32(1.0) + x[0, 0] * jnp.float32(0.0)
    tab = tables.reshape(NUM_LEVELS * TABLE_SIZE * FEAT) * one
    return _hash_encode_sc(x, tab).reshape(N, NUM_LEVELS * FEAT)


# speedup vs baseline: 4.0123x; 3.7976x over previous
import functools

import numpy as np
import jax
import jax.numpy as jnp
from jax import lax
from jax.experimental import pallas as pl
from jax.experimental.pallas import tpu as pltpu
from jax.experimental.pallas import tpu_sc as plsc

NUM_LEVELS = 16
TABLE_SIZE = 2 ** 19
MASK = TABLE_SIZE - 1
FEAT = 2
N = 262144
NC = 2
NS = 16
NW = NC * NS
PER_TILE = N // NW
C = 512
N_CHUNKS = PER_TILE // C
NGROUP = C // 16
GXFER = 128
NXFER = 8 * C // GXFER


def _i32(v: int) -> int:
    v &= 0xFFFFFFFF
    return v - (1 << 32) if v >= (1 << 31) else v


_P1 = _i32(2654435761)
_P2 = _i32(805459861)
_P3 = 3674653429
_SCALES = [float(np.float32(np.float64(1.5) ** l)) for l in range(NUM_LEVELS)]
_KLEV = [_i32(_P3 * l) for l in range(NUM_LEVELS)]


def _hash_encode_sc(x, tab):
    mesh = plsc.VectorSubcoreMesh(core_axis_name="c", subcore_axis_name="s")

    @functools.partial(
        pl.kernel,
        out_type=jax.ShapeDtypeStruct((N * NUM_LEVELS * FEAT,), jnp.float32),
        mesh=mesh,
        compiler_params=pltpu.CompilerParams(needs_layout_passes=False,
                                             use_tc_tiling_on_sc=False),
        scratch_types=[
            pltpu.VMEM((C, 3), jnp.float32),
            pltpu.VMEM((C * NUM_LEVELS * FEAT,), jnp.float32),
            pltpu.VMEM((NXFER, GXFER), jnp.int32),
            pltpu.VMEM((NXFER, GXFER), jnp.int32),
            pltpu.VMEM((NXFER, GXFER), jnp.int32),
            pltpu.VMEM((NXFER, GXFER), jnp.int32),
            pltpu.VMEM((8 * C,), jnp.float32),
            pltpu.VMEM((8 * C,), jnp.float32),
            pltpu.VMEM((8 * C,), jnp.float32),
            pltpu.VMEM((8 * C,), jnp.float32),
            pltpu.SemaphoreType.DMA,
            pltpu.SemaphoreType.DMA,
        ],
    )
    def body(x_hbm, tab_hbm, out_hbm,
             x_v, out_v, ia0, ib0, ia1, ib1, ra0, rb0, ra1, rb1, sem0, sem1):
        i32c = jnp.int32
        wid = lax.axis_index("s") * i32c(NC) + lax.axis_index("c")
        tile_base = wid * i32c(PER_TILE)
        iota = lax.iota(jnp.int32, 16)
        zero_f = jnp.zeros((16,), jnp.float32)
        one_f = jnp.full((16,), 1.0, jnp.float32)
        idx_bufs = ((ia0, ib0), (ia1, ib1))
        rows_bufs = ((ra0, rb0), (ra1, rb1))
        sems = (sem0, sem1)

        def load_x(g, d):
            ridx = jnp.full((16,), g * i32c(16), jnp.int32) + iota
            return plsc.load_gather(x_v, [ridx, jnp.full((16,), d, jnp.int32)])

        def scaled(g, l):
            sc = jnp.full((16,), _SCALES[l], jnp.float32)
            out = []
            for d in range(3):
                xd = load_x(g, d)
                xd = jnp.minimum(jnp.maximum(xd, zero_f), one_f)
                out.append(xd * sc)
            return out

        def pass_a(l, idx_refs):
            klev = jnp.full((16,), _KLEV[l], jnp.int32)
            p1 = jnp.full((16,), _P1, jnp.int32)
            p2 = jnp.full((16,), _P2, jnp.int32)
            mask = jnp.full((16,), MASK, jnp.int32)
            base_l = jnp.full((16,), l * TABLE_SIZE * FEAT, jnp.int32)
            one_i = jnp.full((16,), 1, jnp.int32)
            c127 = jnp.full((16,), 127, jnp.int32)
            c128 = jnp.full((16,), 128, jnp.int32)

            def g_body(g, _):
                sx = scaled(g, l)
                xi = [s.astype(jnp.int32) for s in sx]
                a0 = xi[0]
                a1 = xi[0] + one_i
                mm1 = xi[1] * p1
                m1 = (mm1, mm1 + p1)
                mm2 = xi[2] * p2
                m2 = (mm2 ^ klev, (mm2 + p2) ^ klev)
                t = ((a0 ^ m1[0], a0 ^ m1[1]), (a1 ^ m1[0], a1 ^ m1[1]))
                row = g >> i32c(3)
                col = (g & i32c(7)) * i32c(16)
                for c in range(8):
                    b0, b1, b2 = c & 1, (c >> 1) & 1, (c >> 2) & 1
                    h = (t[b0][b1] ^ m2[b2]) & mask
                    e0 = (h + h + base_l) - (h & c127)
                    r = i32c(c * (C // GXFER)) + row
                    idx_refs[0][r, pl.ds(col, 16)] = e0
                    idx_refs[1][r, pl.ds(col, 16)] = e0 + c128
                return jnp.int32(0)

            lax.fori_loop(jnp.int32(0), jnp.int32(NGROUP), g_body, jnp.int32(0))

        def fire(idx_refs, rows_refs, sem):
            def f_body(j, _):
                off = j * i32c(GXFER)
                for f in range(FEAT):
                    pltpu.async_copy(
                        tab_hbm.at[idx_refs[f].at[j]],
                        rows_refs[f].at[pl.ds(off, GXFER)],
                        sem)
                return jnp.int32(0)

            lax.fori_loop(jnp.int32(0), jnp.int32(NXFER), f_body, jnp.int32(0))

        def drain(idx_refs, rows_refs, sem):
            def d_body(j, _):
                off = j * i32c(GXFER)
                for f in range(FEAT):
                    pltpu.make_async_copy(
                        tab_hbm.at[idx_refs[f].at[j]],
                        rows_refs[f].at[pl.ds(off, GXFER)],
                        sem).wait()
                return jnp.int32(0)

            lax.fori_loop(jnp.int32(0), jnp.int32(NXFER), d_body, jnp.int32(0))

        def pass_b(l, rows_refs):
            def g_body(g, _):
                sx = scaled(g, l)
                xf = [s - s.astype(jnp.int32).astype(jnp.float32) for s in sx]
                w0 = [one_f - f for f in xf]
                wxy = ((w0[0] * w0[1], w0[0] * xf[1]),
                       (xf[0] * w0[1], xf[0] * xf[1]))
                wz = (w0[2], xf[2])
                row0 = jnp.full((16,), g * i32c(16), jnp.int32) + iota
                pbase = (g >> i32c(3)) * i32c(1024) + (g & i32c(7)) * i32c(16)
                acc0 = zero_f
                acc1 = zero_f
                for c in range(8):
                    b0, b1, b2 = c & 1, (c >> 1) & 1, (c >> 2) & 1
                    w = wxy[b0][b1] * wz[b2]
                    ridx = row0 + jnp.full((16,), c * C, jnp.int32)
                    f0 = plsc.load_gather(rows_refs[0], [ridx])
                    f1 = plsc.load_gather(rows_refs[1], [ridx])
                    acc0 = acc0 + w * f0
                    acc1 = acc1 + w * f1
                c0, c1 = 2 * l, 2 * l + 1
                out_v[pl.ds(i32c((c0 >> 3) * 4096 + (c0 & 7) * 128) + pbase,
                            16)] = acc0
                out_v[pl.ds(i32c((c1 >> 3) * 4096 + (c1 & 7) * 128) + pbase,
                            16)] = acc1
                return jnp.int32(0)

            lax.fori_loop(jnp.int32(0), jnp.int32(NGROUP), g_body, jnp.int32(0))

        def chunk_body(ch, _):
            base = tile_base + ch * i32c(C)
            pltpu.sync_copy(x_hbm.at[pl.ds(base, C), :], x_v)
            pass_a(0, idx_bufs[0])
            fire(idx_bufs[0], rows_bufs[0], sems[0])
            for l in range(NUM_LEVELS):
                b = l % 2
                nb = 1 - b
                if l + 1 < NUM_LEVELS:
                    pass_a(l + 1, idx_bufs[nb])
                    fire(idx_bufs[nb], rows_bufs[nb], sems[nb])
                drain(idx_bufs[b], rows_bufs[b], sems[b])
                pass_b(l, rows_bufs[b])
            pblk = base >> i32c(7)
            for cb in range(4):
                pltpu.sync_copy(
                    out_v.at[pl.ds(cb * 4096, 4096)],
                    out_hbm.at[pl.ds(i32c(cb * (1 << 21)) +
                                     pblk * i32c(1024), 4096)])
            return jnp.int32(0)

        lax.fori_loop(jnp.int32(0), jnp.int32(N_CHUNKS), chunk_body, jnp.int32(0))

    return body(x, tab)


def kernel(x, tables):
    tab = (tables.reshape(NUM_LEVELS, TABLE_SIZE // 128, 128, FEAT)
           .transpose(0, 1, 3, 2)
           .reshape(NUM_LEVELS * TABLE_SIZE * FEAT))
    flat = _hash_encode_sc(x, tab)
    out = (flat.reshape(4, N // 128, 8, 128)
           .transpose(1, 3, 0, 2)
           .reshape(N, NUM_LEVELS * FEAT))
    return out


# speedup vs baseline: 12.2747x; 3.0593x over previous
import functools

import numpy as np
import jax
import jax.numpy as jnp
from jax import lax
from jax.experimental import pallas as pl
from jax.experimental.pallas import tpu as pltpu
from jax.experimental.pallas import tpu_sc as plsc

NUM_LEVELS = 16
TABLE_SIZE = 2 ** 19
MASK = TABLE_SIZE - 1
FEAT = 2
N = 262144
NC = 2
NS = 16
NW = NC * NS
PER_TILE = N // NW
C = 512
N_CHUNKS = PER_TILE // C
NGROUP = C // 16
GXFER = 128
NXFER = 8 * C // GXFER


def _i32(v: int) -> int:
    v &= 0xFFFFFFFF
    return v - (1 << 32) if v >= (1 << 31) else v


_P1 = _i32(2654435761)
_P2 = _i32(805459861)
_P3 = 3674653429
_SCALES = [float(np.float32(np.float64(1.5) ** l)) for l in range(NUM_LEVELS)]
_KLEV = [_i32(_P3 * l) for l in range(NUM_LEVELS)]

LUT_LEVELS = 9
_GRID = [int(np.floor(np.float32(_SCALES[l]))) + 2 for l in range(LUT_LEVELS)]
_LUT_BASE = np.cumsum([0] + [s ** 3 for s in _GRID]).tolist()
NLUT = _LUT_BASE[-1]
LUT_ROWS = 480
LUT_PLANE = LUT_ROWS * GXFER // 2
LUT_ROUNDS = LUT_ROWS // NXFER


def _phys_elem(level: int, r: np.ndarray) -> np.ndarray:
    return (level << 20) + ((r >> 7) << 8) + (r & 127)


def _build_lut_indices() -> np.ndarray:
    idx0 = np.zeros(LUT_PLANE, np.int64)
    for l in range(LUT_LEVELS):
        s = _GRID[l]
        zz, yy, xx = np.meshgrid(np.arange(s), np.arange(s), np.arange(s),
                                 indexing="ij")
        h = (xx.astype(np.int64)
             ^ (yy.astype(np.int64) * 2654435761)
             ^ (zz.astype(np.int64) * 805459861)
             ^ (_P3 * l)) & MASK
        idx0[_LUT_BASE[l]:_LUT_BASE[l + 1]] = _phys_elem(l, h.reshape(-1))
    both = np.concatenate([idx0, idx0 + 128]).astype(np.int32)
    return both.reshape(LUT_ROWS, GXFER)


_LUT_IDX = _build_lut_indices()


def _hash_encode_sc(x, tab, lutidx):
    mesh = plsc.VectorSubcoreMesh(core_axis_name="c", subcore_axis_name="s")

    @functools.partial(
        pl.kernel,
        out_type=jax.ShapeDtypeStruct((N * NUM_LEVELS * FEAT,), jnp.float32),
        mesh=mesh,
        compiler_params=pltpu.CompilerParams(needs_layout_passes=False,
                                             use_tc_tiling_on_sc=False),
        scratch_types=[
            pltpu.VMEM((C, 3), jnp.float32),
            pltpu.VMEM((C * NUM_LEVELS * FEAT,), jnp.float32),
            pltpu.VMEM((2 * LUT_PLANE,), jnp.float32),
            pltpu.VMEM((NXFER, GXFER), jnp.int32),
            pltpu.VMEM((NXFER, GXFER), jnp.int32),
            pltpu.VMEM((NXFER, GXFER), jnp.int32),
            pltpu.VMEM((NXFER, GXFER), jnp.int32),
            pltpu.VMEM((8 * C,), jnp.float32),
            pltpu.VMEM((8 * C,), jnp.float32),
            pltpu.VMEM((8 * C,), jnp.float32),
            pltpu.VMEM((8 * C,), jnp.float32),
            pltpu.SemaphoreType.DMA,
            pltpu.SemaphoreType.DMA,
        ],
    )
    def body(x_hbm, tab_hbm, lut_hbm, out_hbm,
             x_v, out_v, lut_v, ia0, ib0, ia1, ib1, ra0, rb0, ra1, rb1,
             sem0, sem1):
        i32c = jnp.int32
        wid = lax.axis_index("s") * i32c(NC) + lax.axis_index("c")
        tile_base = wid * i32c(PER_TILE)
        iota = lax.iota(jnp.int32, 16)
        zero_f = jnp.zeros((16,), jnp.float32)
        one_f = jnp.full((16,), 1.0, jnp.float32)
        idx_bufs = ((ia0, ib0), (ia1, ib1))
        rows_bufs = ((ra0, rb0), (ra1, rb1))
        sems = (sem0, sem1)

        def build_lut():
            def k_body(k, _):
                pltpu.sync_copy(lut_hbm.at[pl.ds(k * i32c(NXFER), NXFER), :],
                                ia0)

                def j_body(j, _):
                    pltpu.async_copy(
                        tab_hbm.at[ia0.at[j]],
                        lut_v.at[pl.ds((k * i32c(NXFER) + j) * i32c(GXFER),
                                       GXFER)],
                        sem0)
                    return jnp.int32(0)

                lax.fori_loop(jnp.int32(0), jnp.int32(NXFER), j_body,
                              jnp.int32(0))

                def w_body(j, _):
                    pltpu.make_async_copy(
                        tab_hbm.at[ia0.at[j]],
                        lut_v.at[pl.ds((k * i32c(NXFER) + j) * i32c(GXFER),
                                       GXFER)],
                        sem0).wait()
                    return jnp.int32(0)

                lax.fori_loop(jnp.int32(0), jnp.int32(NXFER), w_body,
                              jnp.int32(0))
                return jnp.int32(0)

            lax.fori_loop(jnp.int32(0), jnp.int32(LUT_ROUNDS), k_body,
                          jnp.int32(0))

        def load_x(g, d):
            ridx = jnp.full((16,), g * i32c(16), jnp.int32) + iota
            return plsc.load_gather(x_v, [ridx, jnp.full((16,), d, jnp.int32)])

        def scaled(g, l):
            sc = jnp.full((16,), _SCALES[l], jnp.float32)
            out = []
            for d in range(3):
                xd = load_x(g, d)
                xd = jnp.minimum(jnp.maximum(xd, zero_f), one_f)
                out.append(xd * sc)
            return out

        def out_store(l, g, pbase, acc0, acc1):
            c0, c1 = 2 * l, 2 * l + 1
            out_v[pl.ds(i32c((c0 >> 3) * 4096 + (c0 & 7) * 128) + pbase,
                        16)] = acc0
            out_v[pl.ds(i32c((c1 >> 3) * 4096 + (c1 & 7) * 128) + pbase,
                        16)] = acc1

        def trilinear(sx):
            xf = [s - s.astype(jnp.int32).astype(jnp.float32) for s in sx]
            w0 = [one_f - f for f in xf]
            wxy = ((w0[0] * w0[1], w0[0] * xf[1]),
                   (xf[0] * w0[1], xf[0] * xf[1]))
            wz = (w0[2], xf[2])
            return wxy, wz

        def level_lut(l):
            s = _GRID[l]
            lbase = jnp.full((16,), _LUT_BASE[l], jnp.int32)
            sv = jnp.full((16,), s, jnp.int32)
            s2v = jnp.full((16,), s * s, jnp.int32)
            plane = jnp.full((16,), LUT_PLANE, jnp.int32)

            def g_body(g, _):
                sx = scaled(g, l)
                xi = [v.astype(jnp.int32) for v in sx]
                wxy, wz = trilinear(sx)
                cell = lbase + xi[0] + xi[1] * sv + xi[2] * s2v
                pbase = (g >> i32c(3)) * i32c(1024) + (g & i32c(7)) * i32c(16)
                acc0 = zero_f
                acc1 = zero_f
                for c in range(8):
                    b0, b1, b2 = c & 1, (c >> 1) & 1, (c >> 2) & 1
                    w = wxy[b0][b1] * wz[b2]
                    cc = cell + jnp.full((16,), b0 + b1 * s + b2 * s * s,
                                         jnp.int32)
                    f0 = plsc.load_gather(lut_v, [cc])
                    f1 = plsc.load_gather(lut_v, [cc + plane])
                    acc0 = acc0 + w * f0
                    acc1 = acc1 + w * f1
                out_store(l, g, pbase, acc0, acc1)
                return jnp.int32(0)

            lax.fori_loop(jnp.int32(0), jnp.int32(NGROUP), g_body, jnp.int32(0))

        def pass_a(l, idx_refs):
            klev = jnp.full((16,), _KLEV[l], jnp.int32)
            p1 = jnp.full((16,), _P1, jnp.int32)
            p2 = jnp.full((16,), _P2, jnp.int32)
            mask = jnp.full((16,), MASK, jnp.int32)
            base_l = jnp.full((16,), l * TABLE_SIZE * FEAT, jnp.int32)
            one_i = jnp.full((16,), 1, jnp.int32)
            c127 = jnp.full((16,), 127, jnp.int32)
            c128 = jnp.full((16,), 128, jnp.int32)

            def g_body(g, _):
                sx = scaled(g, l)
                xi = [v.astype(jnp.int32) for v in sx]
                a0 = xi[0]
                a1 = xi[0] + one_i
                mm1 = xi[1] * p1
                m1 = (mm1, mm1 + p1)
                mm2 = xi[2] * p2
                m2 = (mm2 ^ klev, (mm2 + p2) ^ klev)
                t = ((a0 ^ m1[0], a0 ^ m1[1]), (a1 ^ m1[0], a1 ^ m1[1]))
                row = g >> i32c(3)
                col = (g & i32c(7)) * i32c(16)
                for c in range(8):
                    b0, b1, b2 = c & 1, (c >> 1) & 1, (c >> 2) & 1
                    h = (t[b0][b1] ^ m2[b2]) & mask
                    e0 = (h + h + base_l) - (h & c127)
                    r = i32c(c * (C // GXFER)) + row
                    idx_refs[0][r, pl.ds(col, 16)] = e0
                    idx_refs[1][r, pl.ds(col, 16)] = e0 + c128
                return jnp.int32(0)

            lax.fori_loop(jnp.int32(0), jnp.int32(NGROUP), g_body, jnp.int32(0))

        def fire(idx_refs, rows_refs, sem):
            def f_body(j, _):
                off = j * i32c(GXFER)
                for f in range(FEAT):
                    pltpu.async_copy(
                        tab_hbm.at[idx_refs[f].at[j]],
                        rows_refs[f].at[pl.ds(off, GXFER)],
                        sem)
                return jnp.int32(0)

            lax.fori_loop(jnp.int32(0), jnp.int32(NXFER), f_body, jnp.int32(0))

        def drain(idx_refs, rows_refs, sem):
            def d_body(j, _):
                off = j * i32c(GXFER)
                for f in range(FEAT):
                    pltpu.make_async_copy(
                        tab_hbm.at[idx_refs[f].at[j]],
                        rows_refs[f].at[pl.ds(off, GXFER)],
                        sem).wait()
                return jnp.int32(0)

            lax.fori_loop(jnp.int32(0), jnp.int32(NXFER), d_body, jnp.int32(0))

        def pass_b(l, rows_refs):
            def g_body(g, _):
                sx = scaled(g, l)
                wxy, wz = trilinear(sx)
                row0 = jnp.full((16,), g * i32c(16), jnp.int32) + iota
                pbase = (g >> i32c(3)) * i32c(1024) + (g & i32c(7)) * i32c(16)
                acc0 = zero_f
                acc1 = zero_f
                for c in range(8):
                    b0, b1, b2 = c & 1, (c >> 1) & 1, (c >> 2) & 1
                    w = wxy[b0][b1] * wz[b2]
                    ridx = row0 + jnp.full((16,), c * C, jnp.int32)
                    f0 = plsc.load_gather(rows_refs[0], [ridx])
                    f1 = plsc.load_gather(rows_refs[1], [ridx])
                    acc0 = acc0 + w * f0
                    acc1 = acc1 + w * f1
                out_store(l, g, pbase, acc0, acc1)
                return jnp.int32(0)

            lax.fori_loop(jnp.int32(0), jnp.int32(NGROUP), g_body, jnp.int32(0))

        build_lut()

        def chunk_body(ch, _):
            base = tile_base + ch * i32c(C)
            pltpu.sync_copy(x_hbm.at[pl.ds(base, C), :], x_v)
            pass_a(LUT_LEVELS, idx_bufs[0])
            fire(idx_bufs[0], rows_bufs[0], sems[0])
            for l in range(LUT_LEVELS):
                level_lut(l)
            for l in range(LUT_LEVELS, NUM_LEVELS):
                b = (l - LUT_LEVELS) % 2
                nb = 1 - b
                if l + 1 < NUM_LEVELS:
                    pass_a(l + 1, idx_bufs[nb])
                    fire(idx_bufs[nb], rows_bufs[nb], sems[nb])
                drain(idx_bufs[b], rows_bufs[b], sems[b])
                pass_b(l, rows_bufs[b])
            pblk = base >> i32c(7)
            for cb in range(4):
                pltpu.sync_copy(
                    out_v.at[pl.ds(cb * 4096, 4096)],
                    out_hbm.at[pl.ds(i32c(cb * (1 << 21)) +
                                     pblk * i32c(1024), 4096)])
            return jnp.int32(0)

        lax.fori_loop(jnp.int32(0), jnp.int32(N_CHUNKS), chunk_body,
                      jnp.int32(0))

    return body(x, tab, lutidx)


def kernel(x, tables):
    tab = (tables.reshape(NUM_LEVELS, TABLE_SIZE // 128, 128, FEAT)
           .transpose(0, 1, 3, 2)
           .reshape(NUM_LEVELS * TABLE_SIZE * FEAT))
    lutidx = jnp.asarray(_LUT_IDX)
    flat = _hash_encode_sc(x, tab, lutidx)
    out = (flat.reshape(4, N // 128, 8, 128)
           .transpose(1, 3, 0, 2)
           .reshape(N, NUM_LEVELS * FEAT))
    return out


# speedup vs baseline: 12.4646x; 1.0155x over previous
import functools

import numpy as np
import jax
import jax.numpy as jnp
from jax import lax
from jax.experimental import pallas as pl
from jax.experimental.pallas import tpu as pltpu
from jax.experimental.pallas import tpu_sc as plsc

NUM_LEVELS = 16
TABLE_SIZE = 2 ** 19
MASK = TABLE_SIZE - 1
FEAT = 2
N = 262144
NC = 2
NS = 16
NW = NC * NS
PER_TILE = N // NW
C = 512
N_CHUNKS = PER_TILE // C
NGROUP = C // 16
GXFER = 128
NXFER = 8 * C // GXFER


def _i32(v: int) -> int:
    v &= 0xFFFFFFFF
    return v - (1 << 32) if v >= (1 << 31) else v


_P1 = _i32(2654435761)
_P2 = _i32(805459861)
_P3 = 3674653429
_SCALES = [float(np.float32(np.float64(1.5) ** l)) for l in range(NUM_LEVELS)]
_KLEV = [_i32(_P3 * l) for l in range(NUM_LEVELS)]

LUT_LEVELS = 9
_GRID = [int(np.floor(np.float32(_SCALES[l]))) + 2 for l in range(LUT_LEVELS)]
_LUT_BASE = np.cumsum([0] + [s ** 3 for s in _GRID]).tolist()
NLUT = _LUT_BASE[-1]
LUT_XFER = 128
LUT_BATCH = 32
LUT_ROWS = 480
LUT_PLANE = LUT_ROWS * LUT_XFER // 2
LUT_ROUNDS = LUT_ROWS // LUT_BATCH


def _phys_elem(level: int, r: np.ndarray) -> np.ndarray:
    return (level << 20) + ((r >> 7) << 8) + (r & 127)


def _build_lut_indices() -> np.ndarray:
    idx0 = np.zeros(LUT_PLANE, np.int64)
    for l in range(LUT_LEVELS):
        s = _GRID[l]
        zz, yy, xx = np.meshgrid(np.arange(s), np.arange(s), np.arange(s),
                                 indexing="ij")
        h = (xx.astype(np.int64)
             ^ (yy.astype(np.int64) * 2654435761)
             ^ (zz.astype(np.int64) * 805459861)
             ^ (_P3 * l)) & MASK
        idx0[_LUT_BASE[l]:_LUT_BASE[l + 1]] = _phys_elem(l, h.reshape(-1))
    both = np.concatenate([idx0, idx0 + 128]).astype(np.int32)
    return both.reshape(LUT_ROWS, LUT_XFER)


_LUT_IDX = _build_lut_indices()


def _hash_encode_sc(x, tab, lutidx):
    mesh = plsc.VectorSubcoreMesh(core_axis_name="c", subcore_axis_name="s")

    @functools.partial(
        pl.kernel,
        out_type=jax.ShapeDtypeStruct((N * NUM_LEVELS * FEAT,), jnp.float32),
        mesh=mesh,
        compiler_params=pltpu.CompilerParams(needs_layout_passes=False,
                                             use_tc_tiling_on_sc=False),
        scratch_types=[
            pltpu.VMEM((C, 3), jnp.float32),
            pltpu.VMEM((3, C), jnp.float32),
            pltpu.VMEM((C * NUM_LEVELS * FEAT,), jnp.float32),
            pltpu.VMEM((2 * LUT_PLANE,), jnp.float32),
            pltpu.VMEM((LUT_BATCH, LUT_XFER), jnp.int32),
            pltpu.VMEM((NXFER, GXFER), jnp.int32),
            pltpu.VMEM((NXFER, GXFER), jnp.int32),
            pltpu.VMEM((NXFER, GXFER), jnp.int32),
            pltpu.VMEM((NXFER, GXFER), jnp.int32),
            pltpu.VMEM((8 * C,), jnp.float32),
            pltpu.VMEM((8 * C,), jnp.float32),
            pltpu.VMEM((8 * C,), jnp.float32),
            pltpu.VMEM((8 * C,), jnp.float32),
            pltpu.SemaphoreType.DMA,
            pltpu.SemaphoreType.DMA,
        ],
    )
    def body(x_hbm, tab_hbm, lut_hbm, out_hbm,
             x_v, x_t, out_v, lut_v, li_v, ia0, ib0, ia1, ib1,
             ra0, rb0, ra1, rb1, sem0, sem1):
        i32c = jnp.int32
        wid = lax.axis_index("s") * i32c(NC) + lax.axis_index("c")
        tile_base = wid * i32c(PER_TILE)
        iota = lax.iota(jnp.int32, 16)
        zero_f = jnp.zeros((16,), jnp.float32)
        one_f = jnp.full((16,), 1.0, jnp.float32)
        idx_bufs = ((ia0, ib0), (ia1, ib1))
        rows_bufs = ((ra0, rb0), (ra1, rb1))
        sems = (sem0, sem1)

        def build_lut():
            def k_body(k, _):
                pltpu.sync_copy(
                    lut_hbm.at[pl.ds(k * i32c(LUT_BATCH), LUT_BATCH), :], li_v)

                def j_body(j, _):
                    pltpu.async_copy(
                        tab_hbm.at[li_v.at[j]],
                        lut_v.at[pl.ds((k * i32c(LUT_BATCH) + j)
                                       * i32c(LUT_XFER), LUT_XFER)],
                        sem0)
                    return jnp.int32(0)

                lax.fori_loop(jnp.int32(0), jnp.int32(LUT_BATCH), j_body,
                              jnp.int32(0))

                def w_body(j, _):
                    pltpu.make_async_copy(
                        tab_hbm.at[li_v.at[j]],
                        lut_v.at[pl.ds((k * i32c(LUT_BATCH) + j)
                                       * i32c(LUT_XFER), LUT_XFER)],
                        sem0).wait()
                    return jnp.int32(0)

                lax.fori_loop(jnp.int32(0), jnp.int32(LUT_BATCH), w_body,
                              jnp.int32(0))
                return jnp.int32(0)

            lax.fori_loop(jnp.int32(0), jnp.int32(LUT_ROUNDS), k_body,
                          jnp.int32(0))

        def prep_x():
            def g_body(g, _):
                ridx = jnp.full((16,), g * i32c(16), jnp.int32) + iota
                off = g * i32c(16)
                for d in range(3):
                    xd = plsc.load_gather(
                        x_v, [ridx, jnp.full((16,), d, jnp.int32)])
                    xd = jnp.minimum(jnp.maximum(xd, zero_f), one_f)
                    x_t[d, pl.ds(off, 16)] = xd
                return jnp.int32(0)

            lax.fori_loop(jnp.int32(0), jnp.int32(NGROUP), g_body, jnp.int32(0))

        def load_xs(g):
            off = g * i32c(16)
            return [x_t[d, pl.ds(off, 16)] for d in range(3)]

        def out_store(l, g, acc0, acc1):
            pbase = (g >> i32c(3)) * i32c(1024) + (g & i32c(7)) * i32c(16)
            c0, c1 = 2 * l, 2 * l + 1
            out_v[pl.ds(i32c((c0 >> 3) * 4096 + (c0 & 7) * 128) + pbase,
                        16)] = acc0
            out_v[pl.ds(i32c((c1 >> 3) * 4096 + (c1 & 7) * 128) + pbase,
                        16)] = acc1

        def trilinear(sx):
            xf = [s - s.astype(jnp.int32).astype(jnp.float32) for s in sx]
            w0 = [one_f - f for f in xf]
            wxy = ((w0[0] * w0[1], w0[0] * xf[1]),
                   (xf[0] * w0[1], xf[0] * xf[1]))
            wz = (w0[2], xf[2])
            return wxy, wz

        def combine_lut(l, g, xs):
            s = _GRID[l]
            sx = [xd * jnp.full((16,), _SCALES[l], jnp.float32) for xd in xs]
            xi = [v.astype(jnp.int32) for v in sx]
            wxy, wz = trilinear(sx)
            cell = (jnp.full((16,), _LUT_BASE[l], jnp.int32) + xi[0]
                    + xi[1] * jnp.full((16,), s, jnp.int32)
                    + xi[2] * jnp.full((16,), s * s, jnp.int32))
            plane = jnp.full((16,), LUT_PLANE, jnp.int32)
            acc0 = zero_f
            acc1 = zero_f
            for c in range(8):
                b0, b1, b2 = c & 1, (c >> 1) & 1, (c >> 2) & 1
                w = wxy[b0][b1] * wz[b2]
                cc = cell + jnp.full((16,), b0 + b1 * s + b2 * s * s,
                                     jnp.int32)
                f0 = plsc.load_gather(lut_v, [cc])
                f1 = plsc.load_gather(lut_v, [cc + plane])
                acc0 = acc0 + w * f0
                acc1 = acc1 + w * f1
            out_store(l, g, acc0, acc1)

        def lut_levels():
            def g_body(g, _):
                xs = load_xs(g)
                for l in range(LUT_LEVELS):
                    combine_lut(l, g, xs)
                return jnp.int32(0)

            lax.fori_loop(jnp.int32(0), jnp.int32(NGROUP), g_body, jnp.int32(0))

        def idx_build(l, g, xs, idx_refs):
            klev = jnp.full((16,), _KLEV[l], jnp.int32)
            p1 = jnp.full((16,), _P1, jnp.int32)
            p2 = jnp.full((16,), _P2, jnp.int32)
            mask = jnp.full((16,), MASK, jnp.int32)
            base_l = jnp.full((16,), l * TABLE_SIZE * FEAT, jnp.int32)
            one_i = jnp.full((16,), 1, jnp.int32)
            c127 = jnp.full((16,), 127, jnp.int32)
            c128 = jnp.full((16,), 128, jnp.int32)
            sx = [xd * jnp.full((16,), _SCALES[l], jnp.float32) for xd in xs]
            xi = [v.astype(jnp.int32) for v in sx]
            a0 = xi[0]
            a1 = xi[0] + one_i
            mm1 = xi[1] * p1
            m1 = (mm1, mm1 + p1)
            mm2 = xi[2] * p2
            m2 = (mm2 ^ klev, (mm2 + p2) ^ klev)
            t = ((a0 ^ m1[0], a0 ^ m1[1]), (a1 ^ m1[0], a1 ^ m1[1]))
            row = g >> i32c(3)
            col = (g & i32c(7)) * i32c(16)
            for c in range(8):
                b0, b1, b2 = c & 1, (c >> 1) & 1, (c >> 2) & 1
                h = (t[b0][b1] ^ m2[b2]) & mask
                e0 = (h + h + base_l) - (h & c127)
                r = i32c(c * (C // GXFER)) + row
                idx_refs[0][r, pl.ds(col, 16)] = e0
                idx_refs[1][r, pl.ds(col, 16)] = e0 + c128

        def pass_a(l, idx_refs):
            def g_body(g, _):
                idx_build(l, g, load_xs(g), idx_refs)
                return jnp.int32(0)

            lax.fori_loop(jnp.int32(0), jnp.int32(NGROUP), g_body, jnp.int32(0))

        def combine_hi(l, g, xs, rows_refs):
            sx = [xd * jnp.full((16,), _SCALES[l], jnp.float32) for xd in xs]
            wxy, wz = trilinear(sx)
            row0 = jnp.full((16,), g * i32c(16), jnp.int32) + iota
            acc0 = zero_f
            acc1 = zero_f
            for c in range(8):
                b0, b1, b2 = c & 1, (c >> 1) & 1, (c >> 2) & 1
                w = wxy[b0][b1] * wz[b2]
                ridx = row0 + jnp.full((16,), c * C, jnp.int32)
                f0 = plsc.load_gather(rows_refs[0], [ridx])
                f1 = plsc.load_gather(rows_refs[1], [ridx])
                acc0 = acc0 + w * f0
                acc1 = acc1 + w * f1
            out_store(l, g, acc0, acc1)

        def hi_pass(l, l_next, rows_refs, idx_refs):
            def g_body(g, _):
                xs = load_xs(g)
                combine_hi(l, g, xs, rows_refs)
                if l_next is not None:
                    idx_build(l_next, g, xs, idx_refs)
                return jnp.int32(0)

            lax.fori_loop(jnp.int32(0), jnp.int32(NGROUP), g_body, jnp.int32(0))

        def fire(idx_refs, rows_refs, sem):
            def f_body(j, _):
                off = j * i32c(GXFER)
                for f in range(FEAT):
                    pltpu.async_copy(
                        tab_hbm.at[idx_refs[f].at[j]],
                        rows_refs[f].at[pl.ds(off, GXFER)],
                        sem)
                return jnp.int32(0)

            lax.fori_loop(jnp.int32(0), jnp.int32(NXFER), f_body, jnp.int32(0))

        def drain(idx_refs, rows_refs, sem):
            def d_body(j, _):
                off = j * i32c(GXFER)
                for f in range(FEAT):
                    pltpu.make_async_copy(
                        tab_hbm.at[idx_refs[f].at[j]],
                        rows_refs[f].at[pl.ds(off, GXFER)],
                        sem).wait()
                return jnp.int32(0)

            lax.fori_loop(jnp.int32(0), jnp.int32(NXFER), d_body, jnp.int32(0))

        build_lut()

        def chunk_body(ch, _):
            base = tile_base + ch * i32c(C)
            pltpu.sync_copy(x_hbm.at[pl.ds(base, C), :], x_v)
            prep_x()
            pass_a(LUT_LEVELS, idx_bufs[0])
            fire(idx_bufs[0], rows_bufs[0], sems[0])
            pass_a(LUT_LEVELS + 1, idx_bufs[1])
            fire(idx_bufs[1], rows_bufs[1], sems[1])
            lut_levels()
            for l in range(LUT_LEVELS, NUM_LEVELS):
                b = (l - LUT_LEVELS) % 2
                drain(idx_bufs[b], rows_bufs[b], sems[b])
                nxt = l + 2 if l + 2 < NUM_LEVELS else None
                hi_pass(l, nxt, rows_bufs[b], idx_bufs[b])
                if nxt is not None:
                    fire(idx_bufs[b], rows_bufs[b], sems[b])
            pblk = base >> i32c(7)
            for cb in range(4):
                pltpu.sync_copy(
                    out_v.at[pl.ds(cb * 4096, 4096)],
                    out_hbm.at[pl.ds(i32c(cb * (1 << 21)) +
                                     pblk * i32c(1024), 4096)])
            return jnp.int32(0)

        lax.fori_loop(jnp.int32(0), jnp.int32(N_CHUNKS), chunk_body,
                      jnp.int32(0))

    return body(x, tab, lutidx)


def kernel(x, tables):
    tab = (tables.reshape(NUM_LEVELS, TABLE_SIZE // 128, 128, FEAT)
           .transpose(0, 1, 3, 2)
           .reshape(NUM_LEVELS * TABLE_SIZE * FEAT))
    lutidx = jnp.asarray(_LUT_IDX)
    flat = _hash_encode_sc(x, tab, lutidx)
    out = (flat.reshape(4, N // 128, 8, 128)
           .transpose(1, 3, 0, 2)
           .reshape(N, NUM_LEVELS * FEAT))
    return out
